# baseline (speedup 1.0000x reference)
import functools

import jax
import jax.numpy as jnp
from jax import lax
from jax.experimental import pallas as pl
from jax.experimental.pallas import tpu as pltpu
from jax.experimental.pallas import tpu_sc as plsc

NC = 2
NS = 16
NW = NC * NS
CH1, NBUF1 = 24, 5
CH2, NBUF2 = 24, 8


def _sc_aggregate(feats, src3, dst3, zrow, n_pad, dp, nch, ch, nbuf):
    rows_tile = n_pad // NS
    mesh = plsc.VectorSubcoreMesh(core_axis_name="c", subcore_axis_name="s")

    @functools.partial(
        pl.kernel,
        mesh=mesh,
        compiler_params=pltpu.CompilerParams(use_tc_tiling_on_sc=False),
        out_type=jax.ShapeDtypeStruct((NC, n_pad, dp), jnp.float32),
        scratch_types=(
            [pltpu.VMEM((nch, ch), jnp.int32)] * 2
            + [pltpu.VMEM((ch, dp), jnp.float32)] * nbuf
            + [pltpu.VMEM_SHARED((n_pad, dp), jnp.float32)]
            + [pltpu.SemaphoreType.DMA] * (2 * nbuf)
        ),
    )
    def k(feats_hbm, src_hbm, dst_hbm, zrow_hbm, out_hbm, sidx, didx, *rest):
        rows = rest[:nbuf]
        acc = rest[nbuf]
        gsem = rest[nbuf + 1:2 * nbuf + 1]
        ssem = rest[2 * nbuf + 1:]
        c = lax.axis_index("c")
        s = lax.axis_index("s")
        wid = s * NC + c

        pltpu.sync_copy(zrow_hbm, acc.at[pl.ds(s * rows_tile, rows_tile)])
        pltpu.sync_copy(src_hbm.at[wid], sidx)
        pltpu.sync_copy(dst_hbm.at[wid], didx)
        plsc.subcore_barrier()

        def g_desc(chk, buf, sem):
            return pltpu.make_async_copy(feats_hbm.at[sidx.at[chk]], buf, sem)

        def s_desc(chk, buf, sem):
            return pltpu.make_async_copy(buf, acc.at[didx.at[chk]], sem)

        def body(g, carry):
            ch0 = nbuf * g
            for b in range(nbuf):
                @pl.when(g > 0)
                def _(b=b):
                    s_desc(ch0 - nbuf + b, rows[b], ssem[b]).wait()

                g_desc(ch0 + b, rows[b], gsem[b]).start()
            for b in range(nbuf):
                g_desc(ch0 + b, rows[b], gsem[b]).wait()
                s_desc(ch0 + b, rows[b], ssem[b]).start(add=True)
            return carry

        lax.fori_loop(0, nch // nbuf, body, 0)
        for b in range(nbuf):
            s_desc(nch - nbuf + b, rows[b], ssem[b]).wait()
        plsc.subcore_barrier()

        pltpu.sync_copy(acc.at[pl.ds(s * rows_tile, rows_tile)],
                        out_hbm.at[c, pl.ds(s * rows_tile, rows_tile)])

    return k(feats, src3, dst3, zrow)


def _edges(edge_index, e, n, n_pad, ch, nbuf):
    unit = NW * ch * nbuf
    e_pad = ((e + unit - 1) // unit) * unit
    nch = e_pad // (NW * ch)
    src_flat, dst_flat = edge_index[0], edge_index[1]
    if e_pad != e:
        j = jnp.arange(e_pad - e, dtype=jnp.int32)
        src_flat = jnp.concatenate([src_flat, j % n])
        dst_flat = jnp.concatenate([dst_flat, n + j % (n_pad - n)])
    return src_flat.reshape(NW, nch, ch), dst_flat.reshape(NW, nch, ch), nch


def _dot_t(a, w_ref):
    return lax.dot_general(a, w_ref[...], (((1,), (1,)), ((), ())),
                           preferred_element_type=jnp.float32)


def _combine1_body(p_ref, x_ref, wl_ref, bl_ref, wr_ref, o_ref, *, d):
    p = p_ref[0] + p_ref[1]
    cnt = jnp.maximum(p[:, d:d + 1], 1.0)
    mean = p[:, :d] / cnt
    y = _dot_t(mean, wl_ref) + _dot_t(x_ref[:, :d], wr_ref) + bl_ref[...]
    o_ref[...] = jnp.maximum(y, 0.0)


def _combine2_body(p_ref, c_ref, h_ref, wl_ref, bl_ref, wr_ref, wo_ref,
                   bo_ref, out_ref, h2_ref, *, d):
    p = p_ref[0] + p_ref[1]
    cnt = jnp.maximum(c_ref[0][:, d:d + 1] + c_ref[1][:, d:d + 1], 1.0)
    mean = p / cnt
    h2 = _dot_t(mean, wl_ref) + _dot_t(h_ref[...], wr_ref) + bl_ref[...]
    h2_ref[...] = h2
    out_ref[...] = _dot_t(h2, wo_ref) + bo_ref[...]


def kernel(x, edge_index, Wl1, bl1, Wr1, Wl2, bl2, Wr2, Wo, bo):
    n, d = x.shape
    e = edge_index.shape[1]
    dp = d + 16
    n_pad = ((n + 8 * NS - 1) // (8 * NS)) * (8 * NS)
    rows_tile = n_pad // NS
    blk = 400
    grid = (n // blk,)

    src3a, dst3a, nch1 = _edges(edge_index, e, n, n_pad, CH1, NBUF1)
    src3b, dst3b, nch2 = _edges(edge_index, e, n, n_pad, CH2, NBUF2)
    x_aug = jnp.concatenate(
        [x, jnp.ones((n, 1), jnp.float32), jnp.zeros((n, dp - d - 1), jnp.float32)],
        axis=1)
    z_dp = jnp.zeros((rows_tile, dp), jnp.float32)
    z_d = jnp.zeros((rows_tile, d), jnp.float32)
    bl1r, bl2r, bor = bl1.reshape(1, d), bl2.reshape(1, d), bo.reshape(1, d)

    wspec = pl.BlockSpec((d, d), lambda i: (0, 0))
    bspec = pl.BlockSpec((1, d), lambda i: (0, 0))
    p1spec = pl.BlockSpec((NC, blk, dp), lambda i: (0, i, 0))
    p2spec = pl.BlockSpec((NC, blk, d), lambda i: (0, i, 0))
    fspec = pl.BlockSpec((blk, d), lambda i: (i, 0))

    part1 = _sc_aggregate(x_aug, src3a, dst3a, z_dp, n_pad, dp, nch1, CH1, NBUF1)
    h = pl.pallas_call(
        functools.partial(_combine1_body, d=d),
        grid=grid,
        in_specs=[p1spec, fspec, wspec, bspec, wspec],
        out_specs=fspec,
        out_shape=jax.ShapeDtypeStruct((n, d), jnp.float32),
    )(part1, x_aug, Wl1, bl1r, Wr1)

    part2 = _sc_aggregate(h, src3b, dst3b, z_d, n_pad, d, nch2, CH2, NBUF2)
    out, h2 = pl.pallas_call(
        functools.partial(_combine2_body, d=d),
        grid=grid,
        in_specs=[p2spec, p1spec, fspec, wspec, bspec, wspec, wspec, bspec],
        out_specs=[fspec, fspec],
        out_shape=[jax.ShapeDtypeStruct((n, d), jnp.float32),
                   jax.ShapeDtypeStruct((n, d), jnp.float32)],
    )(part2, part1, h, Wl2, bl2r, Wr2, Wo, bor)

    return (out, h2)


# speedup vs baseline: 1.0099x; 1.0099x over previous
import functools

import jax
import jax.numpy as jnp
from jax import lax
from jax.experimental import pallas as pl
from jax.experimental.pallas import tpu as pltpu
from jax.experimental.pallas import tpu_sc as plsc

NC = 2
NS = 16
NW = NC * NS
CH1, NBUF1 = 24, 6
CH2, NBUF2 = 40, 5


def _sc_aggregate(feats, src3, dst3, zrow, n_pad, dp, nch, ch, nbuf):
    rows_tile = n_pad // NS
    mesh = plsc.VectorSubcoreMesh(core_axis_name="c", subcore_axis_name="s")

    @functools.partial(
        pl.kernel,
        mesh=mesh,
        compiler_params=pltpu.CompilerParams(use_tc_tiling_on_sc=False),
        out_type=jax.ShapeDtypeStruct((NC, n_pad, dp), jnp.float32),
        scratch_types=(
            [pltpu.VMEM((nch, ch), jnp.int32)] * 2
            + [pltpu.VMEM((ch, dp), jnp.float32)] * nbuf
            + [pltpu.VMEM_SHARED((n_pad, dp), jnp.float32)]
            + [pltpu.SemaphoreType.DMA] * (2 * nbuf)
        ),
    )
    def k(feats_hbm, src_hbm, dst_hbm, zrow_hbm, out_hbm, sidx, didx, *rest):
        rows = rest[:nbuf]
        acc = rest[nbuf]
        gsem = rest[nbuf + 1:2 * nbuf + 1]
        ssem = rest[2 * nbuf + 1:]
        c = lax.axis_index("c")
        s = lax.axis_index("s")
        wid = s * NC + c

        pltpu.sync_copy(zrow_hbm, acc.at[pl.ds(s * rows_tile, rows_tile)])
        pltpu.sync_copy(src_hbm.at[wid], sidx)
        pltpu.sync_copy(dst_hbm.at[wid], didx)
        plsc.subcore_barrier()

        def g_desc(chk, buf, sem):
            return pltpu.make_async_copy(feats_hbm.at[sidx.at[chk]], buf, sem)

        def s_desc(chk, buf, sem):
            return pltpu.make_async_copy(buf, acc.at[didx.at[chk]], sem)

        def body(g, carry):
            ch0 = nbuf * g
            for b in range(nbuf):
                @pl.when(g > 0)
                def _(b=b):
                    s_desc(ch0 - nbuf + b, rows[b], ssem[b]).wait()

                g_desc(ch0 + b, rows[b], gsem[b]).start()
            for b in range(nbuf):
                g_desc(ch0 + b, rows[b], gsem[b]).wait()
                s_desc(ch0 + b, rows[b], ssem[b]).start(add=True)
            return carry

        lax.fori_loop(0, nch // nbuf, body, 0)
        for b in range(nbuf):
            s_desc(nch - nbuf + b, rows[b], ssem[b]).wait()
        plsc.subcore_barrier()

        pltpu.sync_copy(acc.at[pl.ds(s * rows_tile, rows_tile)],
                        out_hbm.at[c, pl.ds(s * rows_tile, rows_tile)])

    return k(feats, src3, dst3, zrow)


def _edges(edge_index, e, n, n_pad, ch, nbuf):
    unit = NW * ch * nbuf
    e_pad = ((e + unit - 1) // unit) * unit
    nch = e_pad // (NW * ch)
    src_flat, dst_flat = edge_index[0], edge_index[1]
    if e_pad != e:
        j = jnp.arange(e_pad - e, dtype=jnp.int32)
        src_flat = jnp.concatenate([src_flat, j % n])
        dst_flat = jnp.concatenate([dst_flat, n + j % (n_pad - n)])
    return src_flat.reshape(NW, nch, ch), dst_flat.reshape(NW, nch, ch), nch


def _dot_t(a, w_ref):
    return lax.dot_general(a, w_ref[...], (((1,), (1,)), ((), ())),
                           preferred_element_type=jnp.float32)


def _combine1_body(p_ref, x_ref, wl_ref, bl_ref, wr_ref, o_ref, *, d):
    p = p_ref[0] + p_ref[1]
    cnt = jnp.maximum(p[:, d:d + 1], 1.0)
    mean = p[:, :d] / cnt
    y = _dot_t(mean, wl_ref) + _dot_t(x_ref[:, :d], wr_ref) + bl_ref[...]
    o_ref[...] = jnp.maximum(y, 0.0)


def _combine2_body(p_ref, c_ref, h_ref, wl_ref, bl_ref, wr_ref, wo_ref,
                   bo_ref, out_ref, h2_ref, *, d):
    p = p_ref[0] + p_ref[1]
    cnt = jnp.maximum(c_ref[0][:, d:d + 1] + c_ref[1][:, d:d + 1], 1.0)
    mean = p / cnt
    h2 = _dot_t(mean, wl_ref) + _dot_t(h_ref[...], wr_ref) + bl_ref[...]
    h2_ref[...] = h2
    out_ref[...] = _dot_t(h2, wo_ref) + bo_ref[...]


def kernel(x, edge_index, Wl1, bl1, Wr1, Wl2, bl2, Wr2, Wo, bo):
    n, d = x.shape
    e = edge_index.shape[1]
    dp = d + 8
    n_pad = ((n + 8 * NS - 1) // (8 * NS)) * (8 * NS)
    rows_tile = n_pad // NS
    blk = 400
    grid = (n // blk,)

    src3a, dst3a, nch1 = _edges(edge_index, e, n, n_pad, CH1, NBUF1)
    src3b, dst3b, nch2 = _edges(edge_index, e, n, n_pad, CH2, NBUF2)
    x_aug = jnp.concatenate(
        [x, jnp.ones((n, 1), jnp.float32), jnp.zeros((n, dp - d - 1), jnp.float32)],
        axis=1)
    z_dp = jnp.zeros((rows_tile, dp), jnp.float32)
    z_d = jnp.zeros((rows_tile, d), jnp.float32)
    bl1r, bl2r, bor = bl1.reshape(1, d), bl2.reshape(1, d), bo.reshape(1, d)

    wspec = pl.BlockSpec((d, d), lambda i: (0, 0))
    bspec = pl.BlockSpec((1, d), lambda i: (0, 0))
    p1spec = pl.BlockSpec((NC, blk, dp), lambda i: (0, i, 0))
    p2spec = pl.BlockSpec((NC, blk, d), lambda i: (0, i, 0))
    fspec = pl.BlockSpec((blk, d), lambda i: (i, 0))

    part1 = _sc_aggregate(x_aug, src3a, dst3a, z_dp, n_pad, dp, nch1, CH1, NBUF1)
    h = pl.pallas_call(
        functools.partial(_combine1_body, d=d),
        grid=grid,
        in_specs=[p1spec, fspec, wspec, bspec, wspec],
        out_specs=fspec,
        out_shape=jax.ShapeDtypeStruct((n, d), jnp.float32),
    )(part1, x_aug, Wl1, bl1r, Wr1)

    part2 = _sc_aggregate(h, src3b, dst3b, z_d, n_pad, d, nch2, CH2, NBUF2)
    out, h2 = pl.pallas_call(
        functools.partial(_combine2_body, d=d),
        grid=grid,
        in_specs=[p2spec, p1spec, fspec, wspec, bspec, wspec, wspec, bspec],
        out_specs=[fspec, fspec],
        out_shape=[jax.ShapeDtypeStruct((n, d), jnp.float32),
                   jax.ShapeDtypeStruct((n, d), jnp.float32)],
    )(part2, part1, h, Wl2, bl2r, Wr2, Wo, bor)

    return (out, h2)


# speedup vs baseline: 1.0757x; 1.0651x over previous
import functools

import jax
import jax.numpy as jnp
from jax import lax
from jax.experimental import pallas as pl
from jax.experimental.pallas import tpu as pltpu
from jax.experimental.pallas import tpu_sc as plsc

NC = 2
NS = 16
NW = NC * NS
CH1, NBUF1 = 24, 6
CH2, NBUF2 = 40, 5


def _sc_aggregate(feats, src3, dst3, zrow, n_pad, dp, nch, ch, nbuf):
    rows_tile = n_pad // NS
    mesh = plsc.VectorSubcoreMesh(core_axis_name="c", subcore_axis_name="s")

    @functools.partial(
        pl.kernel,
        mesh=mesh,
        compiler_params=pltpu.CompilerParams(use_tc_tiling_on_sc=False),
        out_type=jax.ShapeDtypeStruct((NC, n_pad, dp), jnp.float32),
        scratch_types=(
            [pltpu.VMEM((nch, ch), jnp.int32)] * 2
            + [pltpu.VMEM((ch, dp), jnp.float32)] * nbuf
            + [pltpu.VMEM_SHARED((n_pad, dp), jnp.float32)]
            + [pltpu.SemaphoreType.DMA] * (2 * nbuf)
        ),
    )
    def k(feats_hbm, src_hbm, dst_hbm, zrow_hbm, out_hbm, sidx, didx, *rest):
        rows = rest[:nbuf]
        acc = rest[nbuf]
        gsem = rest[nbuf + 1:2 * nbuf + 1]
        ssem = rest[2 * nbuf + 1:]
        c = lax.axis_index("c")
        s = lax.axis_index("s")
        wid = s * NC + c

        pltpu.sync_copy(zrow_hbm, acc.at[pl.ds(s * rows_tile, rows_tile)])
        pltpu.sync_copy(src_hbm.at[wid], sidx)
        pltpu.sync_copy(dst_hbm.at[wid], didx)
        plsc.subcore_barrier()

        def g_desc(chk, buf, sem):
            return pltpu.make_async_copy(feats_hbm.at[sidx.at[chk]], buf, sem)

        def s_desc(chk, buf, sem):
            return pltpu.make_async_copy(buf, acc.at[didx.at[chk]], sem)

        def body(g, carry):
            ch0 = nbuf * g
            for b in range(nbuf):
                @pl.when(g > 0)
                def _(b=b):
                    s_desc(ch0 - nbuf + b, rows[b], ssem[b]).wait()

                g_desc(ch0 + b, rows[b], gsem[b]).start()
            for b in range(nbuf):
                g_desc(ch0 + b, rows[b], gsem[b]).wait()
                s_desc(ch0 + b, rows[b], ssem[b]).start(add=True)
            return carry

        lax.fori_loop(0, nch // nbuf, body, 0)
        for b in range(nbuf):
            s_desc(nch - nbuf + b, rows[b], ssem[b]).wait()
        plsc.subcore_barrier()

        pltpu.sync_copy(acc.at[pl.ds(s * rows_tile, rows_tile)],
                        out_hbm.at[c, pl.ds(s * rows_tile, rows_tile)])

    return k(feats, src3, dst3, zrow)


def _edges(edge_index, e, n, n_pad, ch, nbuf):
    unit = NW * ch * nbuf
    e_pad = ((e + unit - 1) // unit) * unit
    nch = e_pad // (NW * ch)
    src_flat, dst_flat = edge_index[0], edge_index[1]
    if e_pad != e:
        j = jnp.arange(e_pad - e, dtype=jnp.int32)
        src_flat = jnp.concatenate([src_flat, j % n])
        dst_flat = jnp.concatenate([dst_flat, n + j % (n_pad - n)])
    return src_flat.reshape(NW, nch, ch), dst_flat.reshape(NW, nch, ch), nch


def _dot_t(a, w_ref):
    return lax.dot_general(a, w_ref[...], (((1,), (1,)), ((), ())),
                           preferred_element_type=jnp.float32)


def _combine1_body(p_ref, x_ref, wl_ref, bl_ref, wr_ref, o_ref, *, d):
    p = p_ref[0] + p_ref[1]
    cnt = jnp.maximum(p[:, d:d + 1], 1.0)
    mean = p[:, :d] / cnt
    y = _dot_t(mean, wl_ref) + _dot_t(x_ref[:, :d], wr_ref) + bl_ref[...]
    o_ref[...] = jnp.maximum(y, 0.0)


def _combine2_body(p_ref, c_ref, h_ref, wl_ref, bl_ref, wr_ref, wo_ref,
                   bo_ref, out_ref, h2_ref, *, d):
    p = p_ref[0] + p_ref[1]
    cnt = jnp.maximum(c_ref[0][:, d:d + 1] + c_ref[1][:, d:d + 1], 1.0)
    mean = p / cnt
    h2 = _dot_t(mean, wl_ref) + _dot_t(h_ref[...], wr_ref) + bl_ref[...]
    h2_ref[...] = h2
    out_ref[...] = _dot_t(h2, wo_ref) + bo_ref[...]


def kernel(x, edge_index, Wl1, bl1, Wr1, Wl2, bl2, Wr2, Wo, bo):
    n, d = x.shape
    e = edge_index.shape[1]
    dp = d + 8
    n_pad = ((n + 8 * NS - 1) // (8 * NS)) * (8 * NS)
    rows_tile = n_pad // NS
    blk = 2000
    grid = (n // blk,)

    src3a, dst3a, nch1 = _edges(edge_index, e, n, n_pad, CH1, NBUF1)
    src3b, dst3b, nch2 = _edges(edge_index, e, n, n_pad, CH2, NBUF2)
    x_aug = jnp.concatenate(
        [x, jnp.ones((n, 1), jnp.float32), jnp.zeros((n, dp - d - 1), jnp.float32)],
        axis=1)
    z_dp = jnp.zeros((rows_tile, dp), jnp.float32)
    z_d = jnp.zeros((rows_tile, d), jnp.float32)
    bl1r, bl2r, bor = bl1.reshape(1, d), bl2.reshape(1, d), bo.reshape(1, d)

    wspec = pl.BlockSpec((d, d), lambda i: (0, 0))
    bspec = pl.BlockSpec((1, d), lambda i: (0, 0))
    p1spec = pl.BlockSpec((NC, blk, dp), lambda i: (0, i, 0))
    p2spec = pl.BlockSpec((NC, blk, d), lambda i: (0, i, 0))
    fspec = pl.BlockSpec((blk, d), lambda i: (i, 0))

    part1 = _sc_aggregate(x_aug, src3a, dst3a, z_dp, n_pad, dp, nch1, CH1, NBUF1)
    h = pl.pallas_call(
        functools.partial(_combine1_body, d=d),
        grid=grid,
        in_specs=[p1spec, fspec, wspec, bspec, wspec],
        out_specs=fspec,
        out_shape=jax.ShapeDtypeStruct((n, d), jnp.float32),
    )(part1, x_aug, Wl1, bl1r, Wr1)

    part2 = _sc_aggregate(h, src3b, dst3b, z_d, n_pad, d, nch2, CH2, NBUF2)
    out, h2 = pl.pallas_call(
        functools.partial(_combine2_body, d=d),
        grid=grid,
        in_specs=[p2spec, p1spec, fspec, wspec, bspec, wspec, wspec, bspec],
        out_specs=[fspec, fspec],
        out_shape=[jax.ShapeDtypeStruct((n, d), jnp.float32),
                   jax.ShapeDtypeStruct((n, d), jnp.float32)],
    )(part2, part1, h, Wl2, bl2r, Wr2, Wo, bor)

    return (out, h2)
